# baseline (speedup 1.0000x reference)
import dataclasses
import functools

import jax
import jax.numpy as jnp
from jax import lax
from jax.experimental import pallas as pl
from jax.experimental.pallas import tpu as pltpu
from jax.experimental.pallas import tpu_sc as plsc

N_NODES = 10000
D = 128
N_PAD = 10240
N_EDGES = 320000
CHUNK = 88
NC, NS, L = 2, 16, 16
NTILES = NC * NS
SCH = 15
NSB = 16
CPPAIR = NSB * SCH
NSI0 = 10
NSI1 = NSB - NSI0
SEDG = SCH * CHUNK
E_PAD = NS * CPPAIR * CHUNK
N_DUMMY = E_PAD - N_EDGES
ROWS_PER_TILE = N_PAD // NS
COPB = 80
RB = ROWS_PER_TILE // COPB

_SC_MESH = plsc.VectorSubcoreMesh(
    core_axis_name="c", subcore_axis_name="s", num_cores=NC, num_subcores=NS
)

_SC_PARAMS = pltpu.CompilerParams()
if "needs_layout_passes" in pltpu.CompilerParams.__dataclass_fields__:
    _SC_PARAMS = dataclasses.replace(_SC_PARAMS, needs_layout_passes=False)


def _matmul(x_pad, W):
    def body(x_ref, w_ref, o_ref):
        o_ref[...] = jnp.dot(x_ref[...], w_ref[...],
                             preferred_element_type=jnp.float32)

    blk = N_PAD // 8
    return pl.pallas_call(
        body,
        grid=(8,),
        in_specs=[
            pl.BlockSpec((blk, D), lambda i: (i, 0)),
            pl.BlockSpec((D, D), lambda i: (0, 0)),
        ],
        out_specs=pl.BlockSpec((blk, D), lambda i: (i, 0)),
        out_shape=jax.ShapeDtypeStruct((N_PAD, D), jnp.float32),
    )(x_pad, W)


@functools.partial(
    pl.kernel,
    out_type=jax.ShapeDtypeStruct((NC, N_PAD), jnp.float32),
    mesh=_SC_MESH,
    scratch_types=[
        pltpu.VMEM_SHARED((N_PAD,), jnp.float32),
        pltpu.VMEM((SCH, CHUNK), jnp.int32),
        pltpu.VMEM((SEDG,), jnp.float32),
        pltpu.VMEM((ROWS_PER_TILE,), jnp.float32),
        pltpu.SemaphoreType.DMA,
    ],
    compiler_params=_SC_PARAMS,
)
def _deg_kernel(dst2_hbm, ew_hbm, deg_out, deg_sh, dstv, ewv, zbuf, sem):
    c = lax.axis_index("c")
    s = lax.axis_index("s")
    nsi = jnp.where(c == 0, NSI0, NSI1)

    @pl.loop(0, ROWS_PER_TILE // L)
    def _(i):
        zbuf[pl.ds(i * L, L)] = jnp.zeros((L,), jnp.float32)

    pltpu.sync_copy(zbuf, deg_sh.at[pl.ds(s * ROWS_PER_TILE, ROWS_PER_TILE)])
    plsc.subcore_barrier()

    @pl.loop(0, nsi)
    def _(si):
        sb = s * NSB + c * NSI0 + si
        pltpu.sync_copy(dst2_hbm.at[sb], dstv)
        pltpu.sync_copy(ew_hbm.at[pl.ds(sb * SEDG, SEDG)], ewv)

        @pl.loop(0, SCH)
        def _(j):
            pltpu.async_copy(ewv.at[pl.ds(j * CHUNK, CHUNK)],
                             deg_sh.at[dstv.at[j]], sem, add=True)

            @pl.when(j >= 8)
            def _():
                pltpu.make_async_copy(ewv.at[pl.ds(0, CHUNK)],
                                      deg_sh.at[dstv.at[0]], sem).wait()

        @pl.loop(0, 8)
        def _(j):
            pltpu.make_async_copy(ewv.at[pl.ds(0, CHUNK)],
                                  deg_sh.at[dstv.at[0]], sem).wait()

    plsc.subcore_barrier()
    sl = pl.ds(s * ROWS_PER_TILE, ROWS_PER_TILE)
    pltpu.sync_copy(deg_sh.at[sl], deg_out.at[c, sl])


def _dis_kernel(degp):
    def body(d_ref, o_ref):
        o_ref[...] = lax.rsqrt(d_ref[0] + d_ref[1] + 1.0)

    return pl.pallas_call(
        body,
        out_shape=jax.ShapeDtypeStruct((N_PAD // D, D), jnp.float32),
    )(degp)


def _hscale_kernel(h, dis_b):
    def body(h_ref, d_ref, o_ref):
        o_ref[...] = h_ref[...] * d_ref[...]

    blk = N_PAD // 8
    return pl.pallas_call(
        body,
        grid=(8,),
        in_specs=[
            pl.BlockSpec((blk, D), lambda i: (i, 0)),
            pl.BlockSpec((blk, D), lambda i: (i, 0)),
        ],
        out_specs=pl.BlockSpec((blk, D), lambda i: (i, 0)),
        out_shape=jax.ShapeDtypeStruct((N_PAD, D), jnp.float32),
    )(h, dis_b)


@functools.partial(
    pl.kernel,
    out_type=jax.ShapeDtypeStruct((NC, N_PAD, D), jnp.float32),
    mesh=_SC_MESH,
    scratch_types=[
        pltpu.VMEM_SHARED((N_PAD, D), jnp.float32),
        pltpu.VMEM((SEDG,), jnp.int32),
        pltpu.VMEM((SCH, CHUNK), jnp.int32),
        pltpu.VMEM((SEDG,), jnp.float32),
        pltpu.VMEM((SEDG,), jnp.int32),
        pltpu.VMEM((SCH, CHUNK), jnp.int32),
        pltpu.VMEM((SEDG,), jnp.float32),
        pltpu.VMEM((CHUNK, D), jnp.float32),
        pltpu.VMEM((CHUNK, D), jnp.float32),
        pltpu.VMEM((CHUNK, D), jnp.float32),
        pltpu.SemaphoreType.DMA,
        pltpu.SemaphoreType.DMA,
        pltpu.SemaphoreType.DMA,
        pltpu.SemaphoreType.DMA,
        pltpu.SemaphoreType.DMA,
        pltpu.SemaphoreType.DMA,
        pltpu.SemaphoreType.DMA,
        pltpu.SemaphoreType.DMA,
    ],
    compiler_params=_SC_PARAMS,
)
def _agg_kernel(src_hbm, dst2_hbm, ew_hbm, h2_hbm, acc_out,
                acc_sh, srcvA, dstvA, wvA, srcvB, dstvB, wvB,
                rows0, rows1, rows2, g0, g1, g2, s0, s1, s2, lA, lB):
    c = lax.axis_index("c")
    s = lax.axis_index("s")
    nsi = jnp.where(c == 0, NSI0, NSI1)
    rows = (rows0, rows1, rows2)
    gsem = (g0, g1, g2)
    ssem = (s0, s1, s2)
    bufsA = (srcvA, dstvA, wvA, lA)
    bufsB = (srcvB, dstvB, wvB, lB)

    @pl.loop(0, CHUNK)
    def _(i):
        for j in range(D // L):
            rows0[i, pl.ds(j * L, L)] = jnp.zeros((L,), jnp.float32)

    for k in range(RB):
        pltpu.async_copy(
            rows0.at[pl.ds(0, COPB)],
            acc_sh.at[pl.ds(s * ROWS_PER_TILE + k * COPB, COPB)], g0)
    for k in range(RB):
        pltpu.make_async_copy(
            rows0.at[pl.ds(0, COPB)],
            acc_sh.at[pl.ds(s * ROWS_PER_TILE, COPB)], g0).wait()
    plsc.subcore_barrier()

    def fire_gather(srcv, j, u):
        pltpu.async_copy(h2_hbm.at[srcv.at[pl.ds(j * CHUNK, CHUNK)]],
                         rows[u], gsem[u])

    def wait_gather(srcv, u):
        pltpu.make_async_copy(h2_hbm.at[srcv.at[pl.ds(0, CHUNK)]],
                              rows[u], gsem[u]).wait()

    def fire_scatter(dstv, j, u):
        pltpu.async_copy(rows[u], acc_sh.at[dstv.at[j]], ssem[u], add=True)

    def wait_scatter(dstv, u):
        pltpu.make_async_copy(rows[u], acc_sh.at[dstv.at[0]], ssem[u]).wait()

    def scale(wv, j, u):
        buf = rows[u]

        @plsc.parallel_loop(0, CHUNK, unroll=4)
        def _(i):
            idx16 = jnp.zeros((L,), jnp.int32) + (j * CHUNK + i)
            w16 = plsc.load_gather(wv, [idx16])
            for jj in range(D // L):
                sl = pl.ds(jj * L, L)
                buf[i, sl] = buf[i, sl] * w16

    def fire_loads(bufs, si):
        srcv, dstv, wv, lsem = bufs
        sb = s * NSB + c * NSI0 + si
        pltpu.async_copy(src_hbm.at[pl.ds(sb * SEDG, SEDG)], srcv, lsem)
        pltpu.async_copy(dst2_hbm.at[sb], dstv, lsem)
        pltpu.async_copy(ew_hbm.at[pl.ds(sb * SEDG, SEDG)], wv, lsem)

    def wait_loads(bufs):
        srcv, dstv, wv, lsem = bufs
        pltpu.make_async_copy(src_hbm.at[pl.ds(0, SEDG)], srcv, lsem).wait()
        pltpu.make_async_copy(dst2_hbm.at[0], dstv, lsem).wait()
        pltpu.make_async_copy(ew_hbm.at[pl.ds(0, SEDG)], wv, lsem).wait()

    def run_ring(bufs):
        srcv, dstv, wv, _ = bufs
        fire_gather(srcv, 0, 0)

        @pl.loop(0, SCH // 3)
        def _(t):
            j0 = t * 3

            @pl.when(t > 0)
            def _():
                wait_scatter(dstv, 1)
            fire_gather(srcv, j0 + 1, 1)
            wait_gather(srcv, 0)
            scale(wv, j0, 0)
            fire_scatter(dstv, j0, 0)

            @pl.when(t > 0)
            def _():
                wait_scatter(dstv, 2)
            fire_gather(srcv, j0 + 2, 2)
            wait_gather(srcv, 1)
            scale(wv, j0 + 1, 1)
            fire_scatter(dstv, j0 + 1, 1)

            wait_scatter(dstv, 0)

            @pl.when(t < SCH // 3 - 1)
            def _():
                fire_gather(srcv, j0 + 3, 0)
            wait_gather(srcv, 2)
            scale(wv, j0 + 2, 2)
            fire_scatter(dstv, j0 + 2, 2)

        wait_scatter(dstv, 1)
        wait_scatter(dstv, 2)

    fire_loads(bufsA, 0)
    fire_loads(bufsB, 1)

    @pl.loop(0, nsi // 2)
    def _(k):
        wait_loads(bufsA)
        run_ring(bufsA)

        @pl.when(k < nsi // 2 - 1)
        def _():
            fire_loads(bufsA, 2 * k + 2)
        wait_loads(bufsB)
        run_ring(bufsB)

        @pl.when(k < nsi // 2 - 1)
        def _():
            fire_loads(bufsB, 2 * k + 3)

    plsc.subcore_barrier()

    for k in range(RB):
        r0 = s * ROWS_PER_TILE + k * COPB
        pltpu.async_copy(acc_sh.at[pl.ds(r0, COPB)],
                         acc_out.at[c, pl.ds(r0, COPB)], g1)
    for k in range(RB):
        pltpu.make_async_copy(acc_sh.at[pl.ds(s * ROWS_PER_TILE, COPB)],
                              acc_out.at[c, pl.ds(s * ROWS_PER_TILE, COPB)],
                              g1).wait()


def _final_kernel(acc, h2, dis_b, b2d):
    def body(a_ref, h_ref, d_ref, b_ref, o_ref):
        o_ref[...] = jnp.maximum(
            d_ref[...] * (a_ref[0] + a_ref[1] + h_ref[...]) + b_ref[...], 0.0)

    blk = N_NODES // 10
    return pl.pallas_call(
        body,
        grid=(10,),
        in_specs=[
            pl.BlockSpec((NC, blk, D), lambda i: (0, i, 0)),
            pl.BlockSpec((blk, D), lambda i: (i, 0)),
            pl.BlockSpec((blk, D), lambda i: (i, 0)),
            pl.BlockSpec((1, D), lambda i: (0, 0)),
        ],
        out_specs=pl.BlockSpec((blk, D), lambda i: (i, 0)),
        out_shape=jax.ShapeDtypeStruct((N_NODES, D), jnp.float32),
    )(acc, h2, dis_b, b2d)


def kernel(x, edge_index, edge_weights, W, b):
    src = edge_index[0].astype(jnp.int32)
    dst = edge_index[1].astype(jnp.int32)
    ew = edge_weights.astype(jnp.float32)

    dummy_dst = N_NODES + (jnp.arange(N_DUMMY, dtype=jnp.int32) % (N_PAD - N_NODES))
    src_all = jnp.concatenate([src, jnp.full((N_DUMMY,), N_PAD - 1, jnp.int32)])
    dst_all = jnp.concatenate([dst, dummy_dst])
    ew_all = jnp.concatenate([ew, jnp.zeros((N_DUMMY,), jnp.float32)])
    dst2 = dst_all.reshape(E_PAD // SEDG, SCH, CHUNK)

    x_pad = jnp.pad(x.astype(jnp.float32), ((0, N_PAD - N_NODES), (0, 0)))

    h = _matmul(x_pad, W.astype(jnp.float32))
    degp = _deg_kernel(dst2, ew_all)
    dis = _dis_kernel(degp.reshape(NC, N_PAD // D, D))
    dis_b = jnp.broadcast_to(dis.reshape(N_PAD, 1), (N_PAD, D))
    h2 = _hscale_kernel(h, dis_b)
    acc = _agg_kernel(src_all, dst2, ew_all, h2)
    return _final_kernel(acc, h2, dis_b, b.astype(jnp.float32).reshape(1, D))


# speedup vs baseline: 3.5670x; 3.5670x over previous
import dataclasses
import functools

import jax
import jax.numpy as jnp
from jax import lax
from jax.experimental import pallas as pl
from jax.experimental.pallas import tpu as pltpu
from jax.experimental.pallas import tpu_sc as plsc

N_NODES = 10000
D = 128
N_PAD = 10240
N_EDGES = 320000
CHUNK = 96
NC, NS, L = 2, 16, 16
SCH = 27
NSB = 8
CPPAIR = NSB * SCH
NSI0 = 5
NSI1 = NSB - NSI0
SEDG = SCH * CHUNK
E_PAD = NS * CPPAIR * CHUNK
N_DUMMY = E_PAD - N_EDGES - N_PAD
ROWS_PER_TILE = N_PAD // NS
COPB = 80
RB = ROWS_PER_TILE // COPB

_SC_MESH = plsc.VectorSubcoreMesh(
    core_axis_name="c", subcore_axis_name="s", num_cores=NC, num_subcores=NS
)

_SC_PARAMS = pltpu.CompilerParams()
if "needs_layout_passes" in pltpu.CompilerParams.__dataclass_fields__:
    _SC_PARAMS = dataclasses.replace(_SC_PARAMS, needs_layout_passes=False)


def _matmul(x_pad, W):
    def body(x_ref, w_ref, o_ref):
        o_ref[...] = jnp.dot(x_ref[...], w_ref[...],
                             preferred_element_type=jnp.float32)

    blk = N_PAD // 8
    return pl.pallas_call(
        body,
        grid=(8,),
        in_specs=[
            pl.BlockSpec((blk, D), lambda i: (i, 0)),
            pl.BlockSpec((D, D), lambda i: (0, 0)),
        ],
        out_specs=pl.BlockSpec((blk, D), lambda i: (i, 0)),
        out_shape=jax.ShapeDtypeStruct((N_PAD, D), jnp.float32),
    )(x_pad, W)


@functools.partial(
    pl.kernel,
    out_type=jax.ShapeDtypeStruct((NC, N_PAD), jnp.float32),
    mesh=_SC_MESH,
    scratch_types=[
        pltpu.VMEM_SHARED((N_PAD,), jnp.float32),
        pltpu.VMEM((SCH, CHUNK), jnp.int32),
        pltpu.VMEM((SEDG,), jnp.float32),
        pltpu.VMEM((ROWS_PER_TILE,), jnp.float32),
        pltpu.SemaphoreType.DMA,
    ],
    compiler_params=_SC_PARAMS,
)
def _deg_kernel(dst2_hbm, ew_hbm, deg_out, deg_sh, dstv, ewv, zbuf, sem):
    c = lax.axis_index("c")
    s = lax.axis_index("s")
    nsi = jnp.where(c == 0, NSI0, NSI1)

    @pl.loop(0, ROWS_PER_TILE // L)
    def _(i):
        zbuf[pl.ds(i * L, L)] = jnp.zeros((L,), jnp.float32)

    pltpu.sync_copy(zbuf, deg_sh.at[pl.ds(s * ROWS_PER_TILE, ROWS_PER_TILE)])
    plsc.subcore_barrier()

    @pl.loop(0, nsi)
    def _(si):
        sb = s * NSB + c * NSI0 + si
        pltpu.sync_copy(dst2_hbm.at[sb], dstv)
        pltpu.sync_copy(ew_hbm.at[pl.ds(sb * SEDG, SEDG)], ewv)

        @pl.loop(0, SCH)
        def _(j):
            pltpu.async_copy(ewv.at[pl.ds(j * CHUNK, CHUNK)],
                             deg_sh.at[dstv.at[j]], sem, add=True)

            @pl.when(j >= 8)
            def _():
                pltpu.make_async_copy(ewv.at[pl.ds(0, CHUNK)],
                                      deg_sh.at[dstv.at[0]], sem).wait()

        @pl.loop(0, 8)
        def _(j):
            pltpu.make_async_copy(ewv.at[pl.ds(0, CHUNK)],
                                  deg_sh.at[dstv.at[0]], sem).wait()

    plsc.subcore_barrier()
    sl = pl.ds(s * ROWS_PER_TILE, ROWS_PER_TILE)
    pltpu.sync_copy(deg_sh.at[sl], deg_out.at[c, sl])


def _dis_kernel(degp):
    def body(d_ref, o_ref):
        o_ref[...] = lax.rsqrt(d_ref[0] + d_ref[1])

    return pl.pallas_call(
        body,
        out_shape=jax.ShapeDtypeStruct((N_PAD // D, D), jnp.float32),
    )(degp)


def _hscale_kernel(h, dis_b):
    def body(h_ref, d_ref, o_ref):
        o_ref[...] = h_ref[...] * d_ref[...]

    blk = N_PAD // 8
    return pl.pallas_call(
        body,
        grid=(8,),
        in_specs=[
            pl.BlockSpec((blk, D), lambda i: (i, 0)),
            pl.BlockSpec((blk, D), lambda i: (i, 0)),
        ],
        out_specs=pl.BlockSpec((blk, D), lambda i: (i, 0)),
        out_shape=jax.ShapeDtypeStruct((N_PAD, D), jnp.float32),
    )(h, dis_b)


@functools.partial(
    pl.kernel,
    out_type=jax.ShapeDtypeStruct((NC, N_PAD, D), jnp.float32),
    mesh=_SC_MESH,
    scratch_types=[
        pltpu.VMEM_SHARED((N_PAD, D), jnp.float32),
        pltpu.VMEM((SEDG,), jnp.int32),
        pltpu.VMEM((SCH, CHUNK), jnp.int32),
        pltpu.VMEM((SEDG,), jnp.float32),
        pltpu.VMEM((CHUNK, D), jnp.float32),
        pltpu.VMEM((CHUNK, D), jnp.float32),
        pltpu.VMEM((CHUNK, D), jnp.float32),
        pltpu.SemaphoreType.DMA,
        pltpu.SemaphoreType.DMA,
        pltpu.SemaphoreType.DMA,
        pltpu.SemaphoreType.DMA,
        pltpu.SemaphoreType.DMA,
        pltpu.SemaphoreType.DMA,
    ],
    compiler_params=_SC_PARAMS,
)
def _agg_kernel(src_hbm, dst2_hbm, ew_hbm, h2_hbm, acc_out,
                acc_sh, srcv, dstv, wv,
                rows0, rows1, rows2, g0, g1, g2, s0, s1, s2):
    c = lax.axis_index("c")
    s = lax.axis_index("s")
    nsi = jnp.where(c == 0, NSI0, NSI1)
    rows = (rows0, rows1, rows2)
    gsem = (g0, g1, g2)
    ssem = (s0, s1, s2)

    @pl.loop(0, CHUNK)
    def _(i):
        for j in range(D // L):
            rows0[i, pl.ds(j * L, L)] = jnp.zeros((L,), jnp.float32)

    for k in range(RB):
        pltpu.async_copy(
            rows0.at[pl.ds(0, COPB)],
            acc_sh.at[pl.ds(s * ROWS_PER_TILE + k * COPB, COPB)], g0)
    for k in range(RB):
        pltpu.make_async_copy(
            rows0.at[pl.ds(0, COPB)],
            acc_sh.at[pl.ds(s * ROWS_PER_TILE, COPB)], g0).wait()
    plsc.subcore_barrier()

    def fire_gather(j, u):
        pltpu.async_copy(h2_hbm.at[srcv.at[pl.ds(j * CHUNK, CHUNK)]],
                         rows[u], gsem[u])

    def wait_gather(u):
        pltpu.make_async_copy(h2_hbm.at[srcv.at[pl.ds(0, CHUNK)]],
                              rows[u], gsem[u]).wait()

    def fire_scatter(j, u):
        pltpu.async_copy(rows[u], acc_sh.at[dstv.at[j]], ssem[u], add=True)

    def wait_scatter(u):
        pltpu.make_async_copy(rows[u], acc_sh.at[dstv.at[0]], ssem[u]).wait()

    def scale(j, u):
        buf = rows[u]

        @plsc.parallel_loop(0, CHUNK, unroll=4)
        def _(i):
            idx16 = jnp.zeros((L,), jnp.int32) + (j * CHUNK + i)
            w16 = plsc.load_gather(wv, [idx16])
            for jj in range(D // L):
                sl = pl.ds(jj * L, L)
                buf[i, sl] = buf[i, sl] * w16

    @pl.loop(0, nsi)
    def _(si):
        sb = s * NSB + c * NSI0 + si
        sbase = sb * SEDG
        pltpu.sync_copy(src_hbm.at[pl.ds(sbase, SEDG)], srcv)
        pltpu.sync_copy(dst2_hbm.at[sb], dstv)
        pltpu.sync_copy(ew_hbm.at[pl.ds(sbase, SEDG)], wv)

        fire_gather(0, 0)

        @pl.loop(0, SCH // 3)
        def _(t):
            j0 = t * 3

            @pl.when(t > 0)
            def _():
                wait_scatter(1)
            fire_gather(j0 + 1, 1)
            wait_gather(0)
            scale(j0, 0)
            fire_scatter(j0, 0)

            @pl.when(t > 0)
            def _():
                wait_scatter(2)
            fire_gather(j0 + 2, 2)
            wait_gather(1)
            scale(j0 + 1, 1)
            fire_scatter(j0 + 1, 1)

            wait_scatter(0)

            @pl.when(t < SCH // 3 - 1)
            def _():
                fire_gather(j0 + 3, 0)
            wait_gather(2)
            scale(j0 + 2, 2)
            fire_scatter(j0 + 2, 2)

        wait_scatter(1)
        wait_scatter(2)

    plsc.subcore_barrier()

    for k in range(RB):
        r0 = s * ROWS_PER_TILE + k * COPB
        pltpu.async_copy(acc_sh.at[pl.ds(r0, COPB)],
                         acc_out.at[c, pl.ds(r0, COPB)], g1)
    for k in range(RB):
        pltpu.make_async_copy(acc_sh.at[pl.ds(s * ROWS_PER_TILE, COPB)],
                              acc_out.at[c, pl.ds(s * ROWS_PER_TILE, COPB)],
                              g1).wait()


def _final_kernel(acc, dis_b, b2d):
    def body(a_ref, d_ref, b_ref, o_ref):
        o_ref[...] = jnp.maximum(
            d_ref[...] * (a_ref[0] + a_ref[1]) + b_ref[...], 0.0)

    blk = N_NODES // 10
    return pl.pallas_call(
        body,
        grid=(10,),
        in_specs=[
            pl.BlockSpec((NC, blk, D), lambda i: (0, i, 0)),
            pl.BlockSpec((blk, D), lambda i: (i, 0)),
            pl.BlockSpec((1, D), lambda i: (0, 0)),
        ],
        out_specs=pl.BlockSpec((blk, D), lambda i: (i, 0)),
        out_shape=jax.ShapeDtypeStruct((N_NODES, D), jnp.float32),
    )(acc, dis_b, b2d)


def kernel(x, edge_index, edge_weights, W, b):
    src = edge_index[0].astype(jnp.int32)
    dst = edge_index[1].astype(jnp.int32)
    ew = edge_weights.astype(jnp.float32)

    loop_idx = jnp.arange(N_PAD, dtype=jnp.int32)
    dummy_dst = N_NODES + (jnp.arange(N_DUMMY, dtype=jnp.int32) % (N_PAD - N_NODES))
    src_all = jnp.concatenate(
        [src, loop_idx, jnp.full((N_DUMMY,), N_PAD - 1, jnp.int32)])
    dst_all = jnp.concatenate([dst, loop_idx, dummy_dst])
    ew_all = jnp.concatenate(
        [ew, jnp.ones((N_PAD,), jnp.float32), jnp.zeros((N_DUMMY,), jnp.float32)])
    dst2 = dst_all.reshape(E_PAD // SEDG, SCH, CHUNK)

    x_pad = jnp.pad(x.astype(jnp.float32), ((0, N_PAD - N_NODES), (0, 0)))

    h = _matmul(x_pad, W.astype(jnp.float32))
    degp = _deg_kernel(dst2, ew_all)
    dis = _dis_kernel(degp.reshape(NC, N_PAD // D, D))
    dis_b = jnp.broadcast_to(dis.reshape(N_PAD, 1), (N_PAD, D))
    h2 = _hscale_kernel(h, dis_b)
    acc = _agg_kernel(src_all, dst2, ew_all, h2)
    return _final_kernel(acc, dis_b, b.astype(jnp.float32).reshape(1, D))
